# baseline (speedup 1.0000x reference)
import functools

import jax
import jax.numpy as jnp
from jax import lax
from jax.experimental import pallas as pl
from jax.experimental.pallas import tpu as pltpu
from jax.experimental.pallas import tpu_sc as plsc

_N_Q = 4
_N_E = 1024
_E_DIM = 128
_BETA = 0.25
_BM = 3072
_M = 18432
_NBLK = _M // _BM


def _tc_stage_kernel(first, rprev_ref, qprev_ref, w_ref,
                     r_ref, idx_ref, s_ref, wsq_ref):
    i = pl.program_id(0)

    @pl.when(i == 0)
    def _consts():
        w = w_ref[...]
        wsq_ref[0, :] = jnp.sum(w * w, axis=1)

    if first:
        r = rprev_ref[...]
    else:
        r = rprev_ref[...] - qprev_ref[...]
    rsq = jnp.sum(r * r, axis=1, keepdims=True)
    mm = jax.lax.dot_general(r * (-2.0), w_ref[...], (((1,), (1,)), ((), ())),
                             preferred_element_type=jnp.float32)
    score = rsq + mm + wsq_ref[0, :][None, :]
    col_iota = jax.lax.broadcasted_iota(jnp.int32, (_BM, _N_E), 1)
    mn = jnp.min(score, axis=1, keepdims=True)
    idx = jnp.min(jnp.where(score <= mn, col_iota, _N_E), axis=1)
    r_ref[...] = r
    idx_ref[0, 0, :] = idx

    s_blk = jnp.reshape(jnp.sum(rsq), (1, 1))

    @pl.when(i == 0)
    def _s_init():
        s_ref[...] = jnp.zeros((1, 1), jnp.float32)
    s_ref[...] += s_blk


def _tc_stage(first, rprev, qprev, w):
    if first:
        def body(rprev_ref, w_ref, r_ref, idx_ref, s_ref, wsq_ref):
            return _tc_stage_kernel(True, rprev_ref, None, w_ref,
                                    r_ref, idx_ref, s_ref, wsq_ref)
        in_specs = [
            pl.BlockSpec((_BM, _E_DIM), lambda i: (i, 0)),
            pl.BlockSpec((_N_E, _E_DIM), lambda i: (0, 0)),
        ]
        args = (rprev, w)
    else:
        body = functools.partial(_tc_stage_kernel, False)
        in_specs = [
            pl.BlockSpec((_BM, _E_DIM), lambda i: (i, 0)),
            pl.BlockSpec((_BM, _E_DIM), lambda i: (i, 0)),
            pl.BlockSpec((_N_E, _E_DIM), lambda i: (0, 0)),
        ]
        args = (rprev, qprev, w)
    return pl.pallas_call(
        body,
        grid=(_NBLK,),
        in_specs=in_specs,
        out_specs=[
            pl.BlockSpec((_BM, _E_DIM), lambda i: (i, 0)),
            pl.BlockSpec((1, 1, _BM), lambda i: (i, 0, 0)),
            pl.BlockSpec((1, 1), lambda i: (0, 0)),
        ],
        out_shape=[
            jax.ShapeDtypeStruct((_M, _E_DIM), jnp.float32),
            jax.ShapeDtypeStruct((_NBLK, 1, _BM), jnp.int32),
            jax.ShapeDtypeStruct((1, 1), jnp.float32),
        ],
        scratch_shapes=[pltpu.VMEM((1, _N_E), jnp.float32)],
    )(*args)


def _make_sc_gather():
    info = plsc.get_sparse_core_info()
    nw = info.num_cores * info.num_subcores
    b_per_w = _M // nw
    mesh = plsc.VectorSubcoreMesh(core_axis_name="c", subcore_axis_name="s")

    half = b_per_w // 2

    @functools.partial(
        pl.kernel, mesh=mesh,
        out_type=jax.ShapeDtypeStruct((_M, _E_DIM), jnp.float32),
        scratch_types=[
            pltpu.VMEM((b_per_w,), jnp.int32),
            pltpu.VMEM((half, _E_DIM), jnp.float32),
            pltpu.VMEM((half, _E_DIM), jnp.float32),
            pltpu.SemaphoreType.DMA,
            pltpu.SemaphoreType.DMA,
            pltpu.SemaphoreType.DMA,
            pltpu.SemaphoreType.DMA,
        ],
    )
    def gather(table_hbm, idx_hbm, out_hbm, idx_v, rows0, rows1,
               g0, g1, o0, o1):
        wid = lax.axis_index("s") * info.num_cores + lax.axis_index("c")
        base = wid * b_per_w
        pltpu.sync_copy(idx_hbm.at[pl.ds(base, b_per_w)], idx_v)
        cp0 = pltpu.async_copy(table_hbm.at[idx_v.at[pl.ds(0, half)]],
                               rows0, g0)
        cp1 = pltpu.async_copy(table_hbm.at[idx_v.at[pl.ds(half, half)]],
                               rows1, g1)
        cp0.wait()
        oc0 = pltpu.async_copy(rows0, out_hbm.at[pl.ds(base, half)], o0)
        cp1.wait()
        oc1 = pltpu.async_copy(rows1, out_hbm.at[pl.ds(base + half, half)], o1)
        oc0.wait()
        oc1.wait()

    return gather


_sc_gather = _make_sc_gather()


def _tc_final_kernel(z_ref, r3_ref, q3_ref, zq_ref, s_ref):
    i = pl.program_id(0)
    r4 = r3_ref[...] - q3_ref[...]
    zq_ref[...] = z_ref[...] - r4
    s_blk = jnp.reshape(jnp.sum(r4 * r4), (1, 1))

    @pl.when(i == 0)
    def _s_init():
        s_ref[...] = jnp.zeros((1, 1), jnp.float32)
    s_ref[...] += s_blk


def _tc_final(z, r3, q3):
    return pl.pallas_call(
        _tc_final_kernel,
        grid=(_NBLK,),
        in_specs=[pl.BlockSpec((_BM, _E_DIM), lambda i: (i, 0))] * 3,
        out_specs=[
            pl.BlockSpec((_BM, _E_DIM), lambda i: (i, 0)),
            pl.BlockSpec((1, 1), lambda i: (0, 0)),
        ],
        out_shape=[
            jax.ShapeDtypeStruct((_M, _E_DIM), jnp.float32),
            jax.ShapeDtypeStruct((1, 1), jnp.float32),
        ],
    )(z, r3, q3)


def kernel(z, W0, W1, W2, W3):
    B, T, D = z.shape
    zf = z.reshape(_M, D)
    Ws = (W0, W1, W2, W3)
    rprev, qprev = zf, zf
    idxs, s_parts = [], []
    for k in range(_N_Q):
        r, idx, s = _tc_stage(k == 0, rprev, qprev, Ws[k])
        q = _sc_gather(Ws[k], idx.reshape(_M))
        idxs.append(idx.reshape(_M))
        s_parts.append(s[0, 0])
        rprev, qprev = r, q
    zq, s4 = _tc_final(zf, rprev, qprev)
    total_loss = (s_parts[1] + s_parts[2] + s_parts[3] + s4[0, 0]) * (
        (1.0 + _BETA) / (_N_Q * _M * D))
    inds = jnp.stack(idxs, axis=-1).reshape(B, T, _N_Q)
    return (zq.reshape(B, T, D), total_loss, inds)


# speedup vs baseline: 1.0324x; 1.0324x over previous
import functools

import jax
import jax.numpy as jnp
from jax import lax
from jax.experimental import pallas as pl
from jax.experimental.pallas import tpu as pltpu
from jax.experimental.pallas import tpu_sc as plsc

_N_Q = 4
_N_E = 1024
_E_DIM = 128
_BETA = 0.25
_BM = 3072
_M = 18432
_NBLK = _M // _BM


def _tc_stage_kernel(first, rprev_ref, qprev_ref, w_ref,
                     r_ref, idx_ref, s_ref, wsq_ref):
    i = pl.program_id(0)

    @pl.when(i == 0)
    def _consts():
        w = w_ref[...]
        wsq_ref[0, :] = jnp.sum(w * w, axis=1)

    if first:
        r = rprev_ref[...]
    else:
        r = rprev_ref[...] - qprev_ref[...]
    rsq = jnp.sum(r * r, axis=1, keepdims=True)
    mm = jax.lax.dot_general(r * (-2.0), w_ref[...], (((1,), (1,)), ((), ())),
                             preferred_element_type=jnp.float32)
    score = rsq + mm + wsq_ref[0, :][None, :]
    col_iota = jax.lax.broadcasted_iota(jnp.int32, (_BM, _N_E), 1)
    mn = jnp.min(score, axis=1, keepdims=True)
    idx = jnp.min(jnp.where(score <= mn, col_iota, _N_E), axis=1)
    r_ref[...] = r
    idx_ref[0, 0, :] = idx

    s_blk = jnp.reshape(jnp.sum(rsq), (1, 1))

    @pl.when(i == 0)
    def _s_init():
        s_ref[...] = jnp.zeros((1, 1), jnp.float32)
    s_ref[...] += s_blk


def _tc_stage(first, rprev, qprev, w):
    if first:
        def body(rprev_ref, w_ref, r_ref, idx_ref, s_ref, wsq_ref):
            return _tc_stage_kernel(True, rprev_ref, None, w_ref,
                                    r_ref, idx_ref, s_ref, wsq_ref)
        in_specs = [
            pl.BlockSpec((_BM, _E_DIM), lambda i: (i, 0)),
            pl.BlockSpec((_N_E, _E_DIM), lambda i: (0, 0)),
        ]
        args = (rprev, w)
    else:
        body = functools.partial(_tc_stage_kernel, False)
        in_specs = [
            pl.BlockSpec((_BM, _E_DIM), lambda i: (i, 0)),
            pl.BlockSpec((_BM, _E_DIM), lambda i: (i, 0)),
            pl.BlockSpec((_N_E, _E_DIM), lambda i: (0, 0)),
        ]
        args = (rprev, qprev, w)
    return pl.pallas_call(
        body,
        grid=(_NBLK,),
        in_specs=in_specs,
        out_specs=[
            pl.BlockSpec((_BM, _E_DIM), lambda i: (i, 0)),
            pl.BlockSpec((1, 1, _BM), lambda i: (i, 0, 0)),
            pl.BlockSpec((1, 1), lambda i: (0, 0)),
        ],
        out_shape=[
            jax.ShapeDtypeStruct((_M, _E_DIM), jnp.float32),
            jax.ShapeDtypeStruct((_NBLK, 1, _BM), jnp.int32),
            jax.ShapeDtypeStruct((1, 1), jnp.float32),
        ],
        scratch_shapes=[pltpu.VMEM((1, _N_E), jnp.float32)],
    )(*args)


def _make_sc_gather():
    info = plsc.get_sparse_core_info()
    nw = info.num_cores * info.num_subcores
    b_per_w = _M // nw
    mesh = plsc.VectorSubcoreMesh(core_axis_name="c", subcore_axis_name="s")

    @functools.partial(
        pl.kernel, mesh=mesh,
        out_type=jax.ShapeDtypeStruct((_M, _E_DIM), jnp.float32),
        scratch_types=[
            pltpu.VMEM((b_per_w,), jnp.int32),
            pltpu.VMEM((b_per_w, _E_DIM), jnp.float32),
            pltpu.SemaphoreType.DMA,
        ],
    )
    def gather(table_hbm, idx_hbm, out_hbm, idx_v, rows_v, sem):
        wid = lax.axis_index("s") * info.num_cores + lax.axis_index("c")
        base = wid * b_per_w
        pltpu.sync_copy(idx_hbm.at[pl.ds(base, b_per_w)], idx_v)
        pltpu.async_copy(table_hbm.at[idx_v], rows_v, sem).wait()
        pltpu.sync_copy(rows_v, out_hbm.at[pl.ds(base, b_per_w)])

    return gather


_sc_gather = _make_sc_gather()


def _tc_final_kernel(z_ref, r3_ref, q3_ref, zq_ref, s_ref):
    i = pl.program_id(0)
    r4 = r3_ref[...] - q3_ref[...]
    zq_ref[...] = z_ref[...] - r4
    s_blk = jnp.reshape(jnp.sum(r4 * r4), (1, 1))

    @pl.when(i == 0)
    def _s_init():
        s_ref[...] = jnp.zeros((1, 1), jnp.float32)
    s_ref[...] += s_blk


def _tc_final(z, r3, q3):
    return pl.pallas_call(
        _tc_final_kernel,
        grid=(_NBLK,),
        in_specs=[pl.BlockSpec((_BM, _E_DIM), lambda i: (i, 0))] * 3,
        out_specs=[
            pl.BlockSpec((_BM, _E_DIM), lambda i: (i, 0)),
            pl.BlockSpec((1, 1), lambda i: (0, 0)),
        ],
        out_shape=[
            jax.ShapeDtypeStruct((_M, _E_DIM), jnp.float32),
            jax.ShapeDtypeStruct((1, 1), jnp.float32),
        ],
    )(z, r3, q3)


def kernel(z, W0, W1, W2, W3):
    B, T, D = z.shape
    zf = z.reshape(_M, D)
    Ws = (W0, W1, W2, W3)
    rprev, qprev = zf, zf
    idxs, s_parts = [], []
    for k in range(_N_Q):
        r, idx, s = _tc_stage(k == 0, rprev, qprev, Ws[k])
        q = _sc_gather(Ws[k], idx.reshape(_M))
        idxs.append(idx.reshape(_M))
        s_parts.append(s[0, 0])
        rprev, qprev = r, q
    zq, s4 = _tc_final(zf, rprev, qprev)
    total_loss = (s_parts[1] + s_parts[2] + s_parts[3] + s4[0, 0]) * (
        (1.0 + _BETA) / (_N_Q * _M * D))
    inds = jnp.stack(idxs, axis=-1).reshape(B, T, _N_Q)
    return (zq.reshape(B, T, D), total_loss, inds)
